# baseline (speedup 1.0000x reference)
import functools

import jax
import jax.numpy as jnp
from jax import lax
from jax.experimental import pallas as pl
from jax.experimental.pallas import tpu as pltpu
from jax.experimental.pallas import tpu_sc as plsc

_NC = 2
_NS = 16
_NW = _NC * _NS

_B = 32768
_D = 1024
_BPW = _B // _NW
_C = 32
_G = _BPW // _C
_T = _G // 2


def _gather_body(pos_hbm, enc_hbm, out_hbm, idx_v, rows_a, rows_b,
                 gsem_a, gsem_b, ssem_a, ssem_b):
    c = lax.axis_index("c")
    s = lax.axis_index("s")
    wid = s * _NC + c
    base = pl.multiple_of(wid * _BPW, _BPW)

    pltpu.sync_copy(pos_hbm.at[pl.ds(base, _BPW)], idx_v)

    def start_gather(off, buf, sem):
        pltpu.async_copy(enc_hbm.at[idx_v.at[pl.ds(off, _C)]], buf, sem)

    def wait_gather(buf, sem):
        pltpu.make_async_copy(enc_hbm.at[idx_v.at[pl.ds(0, _C)]], buf, sem).wait()

    def start_store(off, buf, sem):
        pltpu.async_copy(buf, out_hbm.at[pl.ds(base + off, _C)], sem)

    def drain_store(buf, sem):
        pltpu.make_async_copy(buf, out_hbm.at[pl.ds(0, _C)], sem).wait()

    start_gather(0, rows_a, gsem_a)
    wait_gather(rows_a, gsem_a)
    start_store(0, rows_a, ssem_a)
    start_gather(_C, rows_b, gsem_b)

    def pair(t, carry):
        off_odd = pl.multiple_of((2 * t + 1) * _C, _C)
        off_even = pl.multiple_of((2 * t + 2) * _C, _C)
        off_next = pl.multiple_of((2 * t + 3) * _C, _C)
        wait_gather(rows_b, gsem_b)
        start_store(off_odd, rows_b, ssem_b)
        drain_store(rows_a, ssem_a)
        start_gather(off_even, rows_a, gsem_a)
        wait_gather(rows_a, gsem_a)
        start_store(off_even, rows_a, ssem_a)
        drain_store(rows_b, ssem_b)
        start_gather(off_next, rows_b, gsem_b)
        return carry

    lax.fori_loop(0, _T - 1, pair, 0)

    off_last = (_G - 1) * _C
    wait_gather(rows_b, gsem_b)
    start_store(off_last, rows_b, ssem_b)
    drain_store(rows_a, ssem_a)
    drain_store(rows_b, ssem_b)


@functools.partial(jax.jit, static_argnames=())
def _gather(pos_flat, encoding):
    mesh = plsc.VectorSubcoreMesh(core_axis_name="c", subcore_axis_name="s")
    run = pl.kernel(
        _gather_body,
        out_type=jax.ShapeDtypeStruct((_B, _D), jnp.float32),
        mesh=mesh,
        scratch_types=(
            pltpu.VMEM((_BPW,), jnp.int32),
            pltpu.VMEM((_C, _D), jnp.float32),
            pltpu.VMEM((_C, _D), jnp.float32),
            pltpu.SemaphoreType.DMA,
            pltpu.SemaphoreType.DMA,
            pltpu.SemaphoreType.DMA,
            pltpu.SemaphoreType.DMA,
        ),
    )
    return run(pos_flat, encoding)


def kernel(pos, encoding):
    b, s = pos.shape
    out = _gather(pos.reshape(-1), encoding)
    return out.reshape(b, s, encoding.shape[1])
